# baseline (speedup 1.0000x reference)
import functools

import jax
import jax.numpy as jnp
from jax import lax
from jax.experimental import pallas as pl
from jax.experimental.pallas import tpu as pltpu
from jax.experimental.pallas import tpu_sc as plsc

_N = 10000
_E = 320000
_C = 40

_NC, _NS, _L = 2, 16, 16
_NW = _NC * _NS
_EW = _E // _NW
_CH = 128
_NCH = 80
_EWP = _NCH * _CH
_NB = 8
_PF = _NB // 2
_NPAD = 10112
_RPT = _NPAD // _NS

_mesh = plsc.VectorSubcoreMesh(core_axis_name="c", subcore_axis_name="s")
_sc_params = pltpu.CompilerParams(use_tc_tiling_on_sc=False)


def _fill_tail(idx_s, idx_d):
    zidx = jnp.zeros((_L,), jnp.int32)
    didx = jnp.full((_L,), _N, jnp.int32)
    for t in range((_EWP - _EW) // _L):
        idx_s[pl.ds(_EW + t * _L, _L)] = zidx
        idx_d[pl.ds(_EW + t * _L, _L)] = didx


@functools.partial(
    pl.kernel,
    out_type=jax.ShapeDtypeStruct((_NC, _NPAD, _L), jnp.float32),
    mesh=_mesh,
    compiler_params=_sc_params,
    scratch_types=[
        pltpu.VMEM((_EWP,), jnp.int32),
        pltpu.VMEM((_CH, _L), jnp.float32),
        pltpu.VMEM((_RPT, _L), jnp.float32),
        pltpu.VMEM_SHARED((_NPAD, _L), jnp.float32),
        pltpu.SemaphoreType.DMA((_NB,)),
    ],
)
def _sc_degree(er, out, idx_d, ones_v, zbuf, agg, ssem):
    c = lax.axis_index("c")
    s = lax.axis_index("s")
    w = c * _NS + s

    def fill_ones(i, _):
        ones_v[i, :] = jnp.ones((_L,), jnp.float32)
        return 0

    lax.fori_loop(0, _CH, fill_ones, 0)

    def fill_zero(i, _):
        zbuf[i, :] = jnp.zeros((_L,), jnp.float32)
        return 0

    lax.fori_loop(0, _RPT, fill_zero, 0)

    didx = jnp.full((_L,), _N, jnp.int32)
    for t in range((_EWP - _EW) // _L):
        idx_d[pl.ds(_EW + t * _L, _L)] = didx

    pltpu.sync_copy(er.at[1, w], idx_d.at[pl.ds(0, _EW)])
    pltpu.sync_copy(zbuf, agg.at[pl.ds(s * _RPT, _RPT)])
    plsc.subcore_barrier()

    def outer(k, _):
        for b in range(_NB):
            j = k * _NB + b
            dsl = idx_d.at[pl.ds(j * _CH, _CH)]

            @pl.when(k > 0)
            def _drain():
                pltpu.make_async_copy(ones_v, agg.at[dsl], ssem.at[b]).wait()

            pltpu.async_copy(ones_v, agg.at[dsl], ssem.at[b], add=True)
        return 0

    lax.fori_loop(0, _NCH // _NB, outer, 0)
    for b in range(_NB):
        pltpu.make_async_copy(ones_v, agg.at[pl.ds(b * _CH, _CH)],
                              ssem.at[b]).wait()
    plsc.subcore_barrier()
    pltpu.sync_copy(agg.at[pl.ds(s * _RPT, _RPT)],
                    out.at[c, pl.ds(s * _RPT, _RPT)])


@functools.partial(
    pl.kernel,
    out_type=jax.ShapeDtypeStruct((_NC, _NPAD, _L), jnp.float32),
    mesh=_mesh,
    compiler_params=_sc_params,
    scratch_types=[
        pltpu.VMEM((_EWP,), jnp.int32),
        pltpu.VMEM((_EWP,), jnp.int32),
        pltpu.VMEM((_NB, _CH, _L), jnp.float32),
        pltpu.VMEM((_RPT, _L), jnp.float32),
        pltpu.VMEM_SHARED((_NPAD, _L), jnp.float32),
        pltpu.SemaphoreType.DMA((_NB,)),
        pltpu.SemaphoreType.DMA((_NB,)),
    ],
)
def _sc_scatter(g, er, out, idx_s, idx_d, rows, zbuf, agg, ssem, gsem):
    c = lax.axis_index("c")
    s = lax.axis_index("s")
    w = c * _NS + s

    def fill_zero(i, _):
        zbuf[i, :] = jnp.zeros((_L,), jnp.float32)
        return 0

    lax.fori_loop(0, _RPT, fill_zero, 0)

    _fill_tail(idx_s, idx_d)
    pltpu.sync_copy(er.at[0, w], idx_s.at[pl.ds(0, _EW)])
    pltpu.sync_copy(er.at[1, w], idx_d.at[pl.ds(0, _EW)])
    pltpu.sync_copy(zbuf, agg.at[pl.ds(s * _RPT, _RPT)])
    plsc.subcore_barrier()

    for b in range(_NB):
        pltpu.async_copy(g.at[idx_s.at[pl.ds(b * _CH, _CH)]], rows.at[b],
                         gsem.at[b])

    def outer(k, _):
        for b in range(_NB):
            j = k * _NB + b
            bn = (b + _PF) % _NB
            dsl = idx_d.at[pl.ds(j * _CH, _CH)]

            @pl.when((j >= _NB - _PF) & (j + _PF < _NCH))
            def _refill():
                pltpu.make_async_copy(rows.at[bn], agg.at[dsl],
                                      ssem.at[bn]).wait()
                pltpu.async_copy(
                    g.at[idx_s.at[pl.ds((j + _PF) * _CH, _CH)]],
                    rows.at[bn], gsem.at[bn])

            pltpu.make_async_copy(g.at[idx_s.at[pl.ds(j * _CH, _CH)]],
                                  rows.at[b], gsem.at[b]).wait()
            pltpu.async_copy(rows.at[b], agg.at[dsl], ssem.at[b], add=True)
        return 0

    lax.fori_loop(0, _NCH // _NB, outer, 0)
    for b in range(_NB):
        pltpu.make_async_copy(rows.at[b], agg.at[pl.ds(b * _CH, _CH)],
                              ssem.at[b]).wait()
    plsc.subcore_barrier()
    pltpu.sync_copy(agg.at[pl.ds(s * _RPT, _RPT)],
                    out.at[c, pl.ds(s * _RPT, _RPT)])


def _tc1_body(x_ref, w1_ref, degp_ref, g_ref, dinv_ref):
    deg = degp_ref[0, :_N, :] + degp_ref[1, :_N, :] + 1.0
    dinv = 1.0 / jnp.sqrt(deg)
    h = jnp.dot(x_ref[...], w1_ref[...], preferred_element_type=jnp.float32)
    dinv_ref[...] = dinv
    g_ref[:_N, :] = h * dinv
    g_ref[_N:, :] = jnp.zeros((_NPAD - _N, 16), jnp.float32)


_tc1 = pl.pallas_call(
    _tc1_body,
    out_shape=(jax.ShapeDtypeStruct((_NPAD, 16), jnp.float32),
               jax.ShapeDtypeStruct((_N, 16), jnp.float32)),
)


def _tc_mid_body(p_ref, g_ref, dinv_ref, b_ref, w_ref, out_ref):
    dinv = dinv_ref[...]
    h = dinv * (p_ref[0, :_N, :] + p_ref[1, :_N, :] + g_ref[:_N, :]) \
        + b_ref[...]
    h = jnp.maximum(h, 0.0)
    out_ref[:_N, :] = jnp.dot(h, w_ref[...],
                              preferred_element_type=jnp.float32) * dinv
    out_ref[_N:, :] = jnp.zeros((_NPAD - _N, 16), jnp.float32)


_tc_mid = pl.pallas_call(
    _tc_mid_body,
    out_shape=jax.ShapeDtypeStruct((_NPAD, 16), jnp.float32),
)


def _leaky(h):
    return jnp.where(h > 0, h, 0.02 * h)


def _tc_fin_body(p_ref, g_ref, dinv_ref, b_ref, m1, mb1, m2, mb2, m3, mb3,
                 out_ref):
    h = dinv_ref[...] * (p_ref[0, :_N, :] + p_ref[1, :_N, :] + g_ref[:_N, :]) \
        + b_ref[...]
    h = _leaky(jnp.dot(h, m1[...], preferred_element_type=jnp.float32)
               + mb1[...])
    h = _leaky(jnp.dot(h, m2[...], preferred_element_type=jnp.float32)
               + mb2[...])
    h = jnp.dot(h, m3[...], preferred_element_type=jnp.float32) + mb3[...]
    mx = jnp.max(h, axis=1, keepdims=True)
    lse = jnp.log(jnp.sum(jnp.exp(h - mx), axis=1, keepdims=True)) + mx
    out_ref[...] = h - lse


_tc_fin = pl.pallas_call(
    _tc_fin_body,
    out_shape=jax.ShapeDtypeStruct((_N, _C), jnp.float32),
)


def kernel(x, edge_index, W1, b1, W2, b2, W3, b3, M1, mb1, M2, mb2, M3, mb3):
    er = edge_index.reshape(2, _NW, _EW)
    degp = _sc_degree(er)
    g1, dinv = _tc1(x, W1, degp)
    p1 = _sc_scatter(g1, er)
    g2 = _tc_mid(p1, g1, dinv, b1.reshape(1, 16), W2)
    p2 = _sc_scatter(g2, er)
    g3 = _tc_mid(p2, g2, dinv, b2.reshape(1, 16), W3)
    p3 = _sc_scatter(g3, er)
    return _tc_fin(p3, g3, dinv, b3.reshape(1, 16), M1, mb1.reshape(1, 64),
                   M2, mb2.reshape(1, 16), M3, mb3.reshape(1, 40))


# speedup vs baseline: 1.4693x; 1.4693x over previous
import functools

import jax
import jax.numpy as jnp
from jax import lax
from jax.experimental import pallas as pl
from jax.experimental.pallas import tpu as pltpu
from jax.experimental.pallas import tpu_sc as plsc

_N = 10000
_E = 320000
_C = 40

_NC, _NS, _L = 2, 16, 16
_NW = _NC * _NS
_EW = _E // _NW
_CH = 128
_NCH = 80
_EWP = _NCH * _CH
_NB = 8
_PF = _NB // 2
_NPAD = 10112
_RPT = _NPAD // _NS

_mesh = plsc.VectorSubcoreMesh(core_axis_name="c", subcore_axis_name="s")
_sc_params = pltpu.CompilerParams(use_tc_tiling_on_sc=False)


def _fill_tail(idx_s, idx_d):
    zidx = jnp.zeros((_L,), jnp.int32)
    didx = jnp.full((_L,), _N, jnp.int32)
    for t in range((_EWP - _EW) // _L):
        idx_s[pl.ds(_EW + t * _L, _L)] = zidx
        idx_d[pl.ds(_EW + t * _L, _L)] = didx


@functools.partial(
    pl.kernel,
    out_type=jax.ShapeDtypeStruct((_NC, _NPAD, _L), jnp.float32),
    mesh=_mesh,
    compiler_params=_sc_params,
    scratch_types=[
        pltpu.VMEM((_EWP,), jnp.int32),
        pltpu.VMEM((_CH, _L), jnp.float32),
        pltpu.VMEM((_RPT, _L), jnp.float32),
        pltpu.VMEM_SHARED((_NPAD, _L), jnp.float32),
        pltpu.SemaphoreType.DMA((_NB,)),
    ],
)
def _sc_degree(er, out, idx_d, ones_v, zbuf, agg, ssem):
    c = lax.axis_index("c")
    s = lax.axis_index("s")
    w = c * _NS + s

    def fill_ones(i, _):
        ones_v[i, :] = jnp.ones((_L,), jnp.float32)
        return 0

    lax.fori_loop(0, _CH, fill_ones, 0)

    def fill_zero(i, _):
        zbuf[i, :] = jnp.zeros((_L,), jnp.float32)
        return 0

    lax.fori_loop(0, _RPT, fill_zero, 0)

    didx = jnp.full((_L,), _N, jnp.int32)
    for t in range((_EWP - _EW) // _L):
        idx_d[pl.ds(_EW + t * _L, _L)] = didx

    pltpu.sync_copy(er.at[1, w], idx_d.at[pl.ds(0, _EW)])
    pltpu.sync_copy(zbuf, agg.at[pl.ds(s * _RPT, _RPT)])
    plsc.subcore_barrier()

    def outer(k, _):
        for b in range(_NB):
            j = k * _NB + b
            dsl = idx_d.at[pl.ds(j * _CH, _CH)]

            @pl.when(k > 0)
            def _drain():
                pltpu.make_async_copy(ones_v, agg.at[dsl], ssem.at[b]).wait()

            pltpu.async_copy(ones_v, agg.at[dsl], ssem.at[b], add=True)
        return 0

    lax.fori_loop(0, _NCH // _NB, outer, 0)
    for b in range(_NB):
        pltpu.make_async_copy(ones_v, agg.at[pl.ds(b * _CH, _CH)],
                              ssem.at[b]).wait()
    plsc.subcore_barrier()
    pltpu.sync_copy(agg.at[pl.ds(s * _RPT, _RPT)],
                    out.at[c, pl.ds(s * _RPT, _RPT)])


@functools.partial(
    pl.kernel,
    out_type=jax.ShapeDtypeStruct((_NC, _NPAD, _L), jnp.float32),
    mesh=_mesh,
    compiler_params=_sc_params,
    scratch_types=[
        pltpu.VMEM((_EWP,), jnp.int32),
        pltpu.VMEM((_EWP,), jnp.int32),
        pltpu.VMEM((_NB, _CH, _L), jnp.float32),
        pltpu.VMEM((_RPT, _L), jnp.float32),
        pltpu.VMEM_SHARED((_NPAD, _L), jnp.float32),
        pltpu.VMEM_SHARED((_NPAD, _L), jnp.float32),
        pltpu.SemaphoreType.DMA((_NB,)),
        pltpu.SemaphoreType.DMA((_NB,)),
    ],
)
def _sc_scatter(g, er, out, idx_s, idx_d, rows, zbuf, gtab, agg, ssem, gsem):
    c = lax.axis_index("c")
    s = lax.axis_index("s")
    w = c * _NS + s

    def fill_zero(i, _):
        zbuf[i, :] = jnp.zeros((_L,), jnp.float32)
        return 0

    lax.fori_loop(0, _RPT, fill_zero, 0)

    _fill_tail(idx_s, idx_d)
    pltpu.sync_copy(er.at[0, w], idx_s.at[pl.ds(0, _EW)])
    pltpu.sync_copy(er.at[1, w], idx_d.at[pl.ds(0, _EW)])
    pltpu.sync_copy(g.at[pl.ds(s * _RPT, _RPT)],
                    gtab.at[pl.ds(s * _RPT, _RPT)])
    pltpu.sync_copy(zbuf, agg.at[pl.ds(s * _RPT, _RPT)])
    plsc.subcore_barrier()

    for b in range(_NB):
        pltpu.async_copy(gtab.at[idx_s.at[pl.ds(b * _CH, _CH)]], rows.at[b],
                         gsem.at[b])

    def outer(k, _):
        for b in range(_NB):
            j = k * _NB + b
            bn = (b + _PF) % _NB
            dsl = idx_d.at[pl.ds(j * _CH, _CH)]

            @pl.when((j >= _NB - _PF) & (j + _PF < _NCH))
            def _refill():
                pltpu.make_async_copy(rows.at[bn], agg.at[dsl],
                                      ssem.at[bn]).wait()
                pltpu.async_copy(
                    gtab.at[idx_s.at[pl.ds((j + _PF) * _CH, _CH)]],
                    rows.at[bn], gsem.at[bn])

            pltpu.make_async_copy(gtab.at[idx_s.at[pl.ds(j * _CH, _CH)]],
                                  rows.at[b], gsem.at[b]).wait()
            pltpu.async_copy(rows.at[b], agg.at[dsl], ssem.at[b], add=True)
        return 0

    lax.fori_loop(0, _NCH // _NB, outer, 0)
    for b in range(_NB):
        pltpu.make_async_copy(rows.at[b], agg.at[pl.ds(b * _CH, _CH)],
                              ssem.at[b]).wait()
    plsc.subcore_barrier()
    pltpu.sync_copy(agg.at[pl.ds(s * _RPT, _RPT)],
                    out.at[c, pl.ds(s * _RPT, _RPT)])


def _tc1_body(x_ref, w1_ref, degp_ref, g_ref, dinv_ref):
    deg = degp_ref[0, :_N, :] + degp_ref[1, :_N, :] + 1.0
    dinv = 1.0 / jnp.sqrt(deg)
    h = jnp.dot(x_ref[...], w1_ref[...], preferred_element_type=jnp.float32)
    dinv_ref[...] = dinv
    g_ref[:_N, :] = h * dinv
    g_ref[_N:, :] = jnp.zeros((_NPAD - _N, 16), jnp.float32)


_tc1 = pl.pallas_call(
    _tc1_body,
    out_shape=(jax.ShapeDtypeStruct((_NPAD, 16), jnp.float32),
               jax.ShapeDtypeStruct((_N, 16), jnp.float32)),
)


def _tc_mid_body(p_ref, g_ref, dinv_ref, b_ref, w_ref, out_ref):
    dinv = dinv_ref[...]
    h = dinv * (p_ref[0, :_N, :] + p_ref[1, :_N, :] + g_ref[:_N, :]) \
        + b_ref[...]
    h = jnp.maximum(h, 0.0)
    out_ref[:_N, :] = jnp.dot(h, w_ref[...],
                              preferred_element_type=jnp.float32) * dinv
    out_ref[_N:, :] = jnp.zeros((_NPAD - _N, 16), jnp.float32)


_tc_mid = pl.pallas_call(
    _tc_mid_body,
    out_shape=jax.ShapeDtypeStruct((_NPAD, 16), jnp.float32),
)


def _leaky(h):
    return jnp.where(h > 0, h, 0.02 * h)


def _tc_fin_body(p_ref, g_ref, dinv_ref, b_ref, m1, mb1, m2, mb2, m3, mb3,
                 out_ref):
    h = dinv_ref[...] * (p_ref[0, :_N, :] + p_ref[1, :_N, :] + g_ref[:_N, :]) \
        + b_ref[...]
    h = _leaky(jnp.dot(h, m1[...], preferred_element_type=jnp.float32)
               + mb1[...])
    h = _leaky(jnp.dot(h, m2[...], preferred_element_type=jnp.float32)
               + mb2[...])
    h = jnp.dot(h, m3[...], preferred_element_type=jnp.float32) + mb3[...]
    mx = jnp.max(h, axis=1, keepdims=True)
    lse = jnp.log(jnp.sum(jnp.exp(h - mx), axis=1, keepdims=True)) + mx
    out_ref[...] = h - lse


_tc_fin = pl.pallas_call(
    _tc_fin_body,
    out_shape=jax.ShapeDtypeStruct((_N, _C), jnp.float32),
)


def kernel(x, edge_index, W1, b1, W2, b2, W3, b3, M1, mb1, M2, mb2, M3, mb3):
    er = edge_index.reshape(2, _NW, _EW)
    degp = _sc_degree(er)
    g1, dinv = _tc1(x, W1, degp)
    p1 = _sc_scatter(g1, er)
    g2 = _tc_mid(p1, g1, dinv, b1.reshape(1, 16), W2)
    p2 = _sc_scatter(g2, er)
    g3 = _tc_mid(p2, g2, dinv, b2.reshape(1, 16), W3)
    p3 = _sc_scatter(g3, er)
    return _tc_fin(p3, g3, dinv, b3.reshape(1, 16), M1, mb1.reshape(1, 64),
                   M2, mb2.reshape(1, 16), M3, mb3.reshape(1, 40))
